# baseline (speedup 1.0000x reference)
import functools

import jax
import jax.numpy as jnp
from jax import lax
from jax.experimental import pallas as pl
from jax.experimental.pallas import tpu as pltpu
from jax.experimental.pallas import tpu_sc as plsc

_NC = 2
_NS = 16
_NW = _NC * _NS
_CHUNK = 128
_PK = 4


def _make_sc_gather(B, D):
    assert B % (_NW * _CHUNK) == 0
    b_per_w = B // _NW
    chunks = b_per_w // _CHUNK
    W = _PK * D
    mesh = plsc.VectorSubcoreMesh(core_axis_name="c", subcore_axis_name="s")

    @functools.partial(
        pl.kernel,
        mesh=mesh,
        out_type=[
            jax.ShapeDtypeStruct((B, D), jnp.float32),
            jax.ShapeDtypeStruct((B, D), jnp.float32),
        ],
        scratch_types=[
            pltpu.VMEM((chunks, _CHUNK), jnp.int32),
            pltpu.VMEM((chunks, _CHUNK), jnp.int32),
            pltpu.VMEM((2, _CHUNK, W), jnp.float32),
            pltpu.VMEM((b_per_w, D), jnp.float32),
            pltpu.SemaphoreType.DMA,
            pltpu.SemaphoreType.DMA,
        ],
    )
    def sc_gather(uhi_hbm, uoff_hbm, mhi_hbm, moff_hbm, utab_hbm, mtab_hbm,
                  u_out, m_out, hi_v, off_v, slabs_v, rows_v,
                  sem0, sem1):
        wid = lax.axis_index("s") * _NC + lax.axis_index("c")
        row0 = wid * chunks
        base = wid * b_per_w
        sems = (sem0, sem1)

        for hi_hbm, off_hbm, tab_hbm, out in (
                (uhi_hbm, uoff_hbm, utab_hbm, u_out),
                (mhi_hbm, moff_hbm, mtab_hbm, m_out)):
            pltpu.sync_copy(hi_hbm.at[pl.ds(row0, chunks)], hi_v)
            pltpu.sync_copy(off_hbm.at[pl.ds(row0, chunks)], off_v)

            def fire(j):
                return pltpu.async_copy(tab_hbm.at[hi_v.at[j]],
                                        slabs_v.at[j % 2], sems[j % 2])

            handles = [None] * chunks
            for j in range(min(2, chunks)):
                handles[j] = fire(j)
            for j in range(chunks):
                handles[j].wait()
                buf = slabs_v.at[j % 2]

                def extract(g, _):
                    offs = off_v[j, pl.ds(g * 16, 16)]
                    for t in range(16):
                        i = g * 16 + t
                        off = offs[t]
                        row = j * _CHUNK + i
                        rows_v[row, pl.ds(0, 16)] = buf[i, pl.ds(off, 16)]
                        rows_v[row, pl.ds(16, 16)] = buf[i, pl.ds(off + 16, 16)]
                    return 0

                lax.fori_loop(0, _CHUNK // 16, extract, 0)
                if j + 2 < chunks:
                    handles[j + 2] = fire(j + 2)
            pltpu.sync_copy(rows_v, out.at[pl.ds(base, b_per_w)])

    return sc_gather


def _dense_body(u_ref, m_ref, wu_ref, bu_ref, wm_ref, bm_ref, o_ref):
    du = jnp.dot(u_ref[...], wu_ref[...],
                 preferred_element_type=jnp.float32) + bu_ref[...]
    dm = jnp.dot(m_ref[...], wm_ref[...],
                 preferred_element_type=jnp.float32) + bm_ref[...]
    o_ref[...] = jnp.sum(du * dm, axis=1, keepdims=True)


def kernel(user_input, movie_input, user_emb, item_emb, Wu, bu, Wm, bm):
    B = user_input.shape[0]
    K = user_emb.shape[1]
    H = Wu.shape[1]

    uidx = user_input.reshape(B // _CHUNK, _CHUNK)
    midx = movie_input.reshape(B // _CHUNK, _CHUNK)
    uhi, uoff = uidx >> 2, (uidx & (_PK - 1)) * K
    mhi, moff = midx >> 2, (midx & (_PK - 1)) * K
    utab = user_emb.reshape(user_emb.shape[0] // _PK, _PK * K)
    mtab = item_emb.reshape(item_emb.shape[0] // _PK, _PK * K)
    u_rows, m_rows = _make_sc_gather(B, K)(uhi, uoff, mhi, moff, utab, mtab)

    BLK = 2048
    z = pl.pallas_call(
        _dense_body,
        grid=(B // BLK,),
        in_specs=[
            pl.BlockSpec((BLK, K), lambda i: (i, 0)),
            pl.BlockSpec((BLK, K), lambda i: (i, 0)),
            pl.BlockSpec((K, H), lambda i: (0, 0)),
            pl.BlockSpec((1, H), lambda i: (0, 0)),
            pl.BlockSpec((K, H), lambda i: (0, 0)),
            pl.BlockSpec((1, H), lambda i: (0, 0)),
        ],
        out_specs=pl.BlockSpec((BLK, 1), lambda i: (i, 0)),
        out_shape=jax.ShapeDtypeStruct((B, 1), jnp.float32),
    )(u_rows, m_rows, Wu, bu.reshape(1, H), Wm, bm.reshape(1, H))
    return z


# speedup vs baseline: 1.0005x; 1.0005x over previous
import functools

import jax
import jax.numpy as jnp
from jax import lax
from jax.experimental import pallas as pl
from jax.experimental.pallas import tpu as pltpu
from jax.experimental.pallas import tpu_sc as plsc

_NC = 2
_NS = 16
_NW = _NC * _NS
_CHUNK = 128
_PK = 4


def _make_sc_gather(B, D):
    assert B % (_NW * _CHUNK) == 0
    b_per_w = B // _NW
    chunks = b_per_w // _CHUNK
    W = _PK * D
    mesh = plsc.VectorSubcoreMesh(core_axis_name="c", subcore_axis_name="s")

    @functools.partial(
        pl.kernel,
        mesh=mesh,
        out_type=[
            jax.ShapeDtypeStruct((B, D), jnp.float32),
            jax.ShapeDtypeStruct((B, D), jnp.float32),
        ],
        scratch_types=[
            pltpu.VMEM((chunks, _CHUNK), jnp.int32),
            pltpu.VMEM((chunks, _CHUNK), jnp.int32),
            pltpu.VMEM((2, _CHUNK, W), jnp.float32),
            pltpu.VMEM((b_per_w, D), jnp.float32),
            pltpu.SemaphoreType.DMA,
            pltpu.SemaphoreType.DMA,
        ],
        compiler_params=pltpu.CompilerParams(use_tc_tiling_on_sc=True),
    )
    def sc_gather(uhi_hbm, uoff_hbm, mhi_hbm, moff_hbm, utab_hbm, mtab_hbm,
                  u_out, m_out, hi_v, off_v, slabs_v, rows_v,
                  sem0, sem1):
        wid = lax.axis_index("s") * _NC + lax.axis_index("c")
        row0 = wid * chunks
        base = wid * b_per_w
        sems = (sem0, sem1)

        for hi_hbm, off_hbm, tab_hbm, out in (
                (uhi_hbm, uoff_hbm, utab_hbm, u_out),
                (mhi_hbm, moff_hbm, mtab_hbm, m_out)):
            pltpu.sync_copy(hi_hbm.at[pl.ds(row0, chunks)], hi_v)
            pltpu.sync_copy(off_hbm.at[pl.ds(row0, chunks)], off_v)

            def fire(j):
                return pltpu.async_copy(tab_hbm.at[hi_v.at[j]],
                                        slabs_v.at[j % 2], sems[j % 2])

            handles = [None] * chunks
            for j in range(min(2, chunks)):
                handles[j] = fire(j)
            for j in range(chunks):
                handles[j].wait()
                buf = slabs_v.at[j % 2]

                def extract(g, _):
                    offs = off_v[j, pl.ds(g * 16, 16)]
                    for t in range(16):
                        i = g * 16 + t
                        off = offs[t]
                        row = j * _CHUNK + i
                        rows_v[row, pl.ds(0, 16)] = buf[i, pl.ds(off, 16)]
                        rows_v[row, pl.ds(16, 16)] = buf[i, pl.ds(off + 16, 16)]
                    return 0

                lax.fori_loop(0, _CHUNK // 16, extract, 0)
                if j + 2 < chunks:
                    handles[j + 2] = fire(j + 2)
            pltpu.sync_copy(rows_v, out.at[pl.ds(base, b_per_w)])

    return sc_gather


def _dense_body(u_ref, m_ref, wu_ref, bu_ref, wm_ref, bm_ref, o_ref):
    du = jnp.dot(u_ref[...], wu_ref[...],
                 preferred_element_type=jnp.float32) + bu_ref[...]
    dm = jnp.dot(m_ref[...], wm_ref[...],
                 preferred_element_type=jnp.float32) + bm_ref[...]
    o_ref[...] = jnp.sum(du * dm, axis=1, keepdims=True)


def kernel(user_input, movie_input, user_emb, item_emb, Wu, bu, Wm, bm):
    B = user_input.shape[0]
    K = user_emb.shape[1]
    H = Wu.shape[1]

    uidx = user_input.reshape(B // _CHUNK, _CHUNK)
    midx = movie_input.reshape(B // _CHUNK, _CHUNK)
    uhi, uoff = uidx >> 2, (uidx & (_PK - 1)) * K
    mhi, moff = midx >> 2, (midx & (_PK - 1)) * K
    utab = user_emb.reshape(user_emb.shape[0] // _PK, _PK * K)
    mtab = item_emb.reshape(item_emb.shape[0] // _PK, _PK * K)
    u_rows, m_rows = _make_sc_gather(B, K)(uhi, uoff, mhi, moff, utab, mtab)

    BLK = 2048
    z = pl.pallas_call(
        _dense_body,
        grid=(B // BLK,),
        in_specs=[
            pl.BlockSpec((BLK, K), lambda i: (i, 0)),
            pl.BlockSpec((BLK, K), lambda i: (i, 0)),
            pl.BlockSpec((K, H), lambda i: (0, 0)),
            pl.BlockSpec((1, H), lambda i: (0, 0)),
            pl.BlockSpec((K, H), lambda i: (0, 0)),
            pl.BlockSpec((1, H), lambda i: (0, 0)),
        ],
        out_specs=pl.BlockSpec((BLK, 1), lambda i: (i, 0)),
        out_shape=jax.ShapeDtypeStruct((B, 1), jnp.float32),
    )(u_rows, m_rows, Wu, bu.reshape(1, H), Wm, bm.reshape(1, H))
    return z


# speedup vs baseline: 1.3892x; 1.3885x over previous
import functools

import jax
import jax.numpy as jnp
from jax import lax
from jax.experimental import pallas as pl
from jax.experimental.pallas import tpu as pltpu
from jax.experimental.pallas import tpu_sc as plsc

_NC = 2
_NS = 16
_NW = _NC * _NS
_CHUNK = 128
_G = 8
_HP = 16
_BLKN = 4096


def _proj_body(t_ref, w_ref, b_ref, o_ref):
    p = lax.dot_general(t_ref[...], w_ref[...], (((0,), (0,)), ((), ())),
                        preferred_element_type=jnp.float32) + b_ref[...]
    p3 = p.reshape(_BLKN // _G, _G, _HP)
    o_ref[...] = jnp.concatenate([p3[:, j, :] for j in range(_G)], axis=-1)


def _project(tab_t, w, b, nsteps):
    K = tab_t.shape[0]
    return pl.pallas_call(
        _proj_body,
        grid=(nsteps,),
        in_specs=[
            pl.BlockSpec((K, _BLKN), lambda i: (0, i)),
            pl.BlockSpec((K, _HP), lambda i: (0, 0)),
            pl.BlockSpec((1, _HP), lambda i: (0, 0)),
        ],
        out_specs=pl.BlockSpec((_BLKN // _G, _G * _HP), lambda i: (i, 0)),
        out_shape=jax.ShapeDtypeStruct((nsteps * _BLKN // _G, _G * _HP),
                                       jnp.float32),
    )(tab_t, w, b)


def _make_sc_zdot(B, H):
    assert B % (_NW * _CHUNK) == 0
    b_per_w = B // _NW
    chunks = b_per_w // _CHUNK
    W = _G * _HP
    mesh = plsc.VectorSubcoreMesh(core_axis_name="c", subcore_axis_name="s")

    @functools.partial(
        pl.kernel,
        mesh=mesh,
        out_type=jax.ShapeDtypeStruct((B,), jnp.float32),
        scratch_types=[
            pltpu.VMEM((chunks, _CHUNK), jnp.int32),
            pltpu.VMEM((chunks, _CHUNK), jnp.int32),
            pltpu.VMEM((chunks, _CHUNK), jnp.int32),
            pltpu.VMEM((chunks, _CHUNK), jnp.int32),
            pltpu.VMEM((2, _CHUNK, W), jnp.float32),
            pltpu.VMEM((2, _CHUNK, W), jnp.float32),
            pltpu.VMEM((b_per_w,), jnp.float32),
            pltpu.SemaphoreType.DMA,
            pltpu.SemaphoreType.DMA,
            pltpu.SemaphoreType.DMA,
            pltpu.SemaphoreType.DMA,
        ],
        compiler_params=pltpu.CompilerParams(needs_layout_passes=False),
    )
    def sc_zdot(uhi_hbm, uoff_hbm, mhi_hbm, moff_hbm, utab_hbm, mtab_hbm,
                z_out, uhi_v, uoff_v, mhi_v, moff_v, ubuf, mbuf, z_v,
                us0, us1, ms0, ms1):
        wid = lax.axis_index("s") * _NC + lax.axis_index("c")
        row0 = wid * chunks
        base = wid * b_per_w
        usems = (us0, us1)
        msems = (ms0, ms1)

        pltpu.sync_copy(uhi_hbm.at[pl.ds(row0, chunks)], uhi_v)
        pltpu.sync_copy(uoff_hbm.at[pl.ds(row0, chunks)], uoff_v)
        pltpu.sync_copy(mhi_hbm.at[pl.ds(row0, chunks)], mhi_v)
        pltpu.sync_copy(moff_hbm.at[pl.ds(row0, chunks)], moff_v)

        def fire(j):
            return (
                pltpu.async_copy(utab_hbm.at[uhi_v.at[j]],
                                 ubuf.at[j % 2], usems[j % 2]),
                pltpu.async_copy(mtab_hbm.at[mhi_v.at[j]],
                                 mbuf.at[j % 2], msems[j % 2]),
            )

        lane = lax.iota(jnp.int32, 16)
        handles = [None] * chunks
        for j in range(min(2, chunks)):
            handles[j] = fire(j)
        for j in range(chunks):
            hu, hm = handles[j]
            hu.wait()
            hm.wait()
            ub = ubuf.at[j % 2]
            mb = mbuf.at[j % 2]

            def compute(g, _):
                rows = g * 16 + lane
                uo = uoff_v[j, pl.ds(g * 16, 16)]
                mo = moff_v[j, pl.ds(g * 16, 16)]
                acc = (plsc.load_gather(ub, [rows, uo])
                       * plsc.load_gather(mb, [rows, mo]))
                for o in range(1, 10):
                    acc = acc + (plsc.load_gather(ub, [rows, uo + o])
                                 * plsc.load_gather(mb, [rows, mo + o]))
                z_v[pl.ds(j * _CHUNK + g * 16, 16)] = acc
                return 0

            lax.fori_loop(0, _CHUNK // 16, compute, 0)
            if j + 2 < chunks:
                handles[j + 2] = fire(j + 2)
        pltpu.sync_copy(z_v, z_out.at[pl.ds(base, b_per_w)])

    return sc_zdot


def kernel(user_input, movie_input, user_emb, item_emb, Wu, bu, Wm, bm):
    B = user_input.shape[0]
    K = user_emb.shape[1]
    H = Wu.shape[1]

    wu = jnp.zeros((K, _HP), jnp.float32).at[:, :H].set(Wu)
    wm = jnp.zeros((K, _HP), jnp.float32).at[:, :H].set(Wm)
    bup = jnp.zeros((1, _HP), jnp.float32).at[0, :H].set(bu)
    bmp = jnp.zeros((1, _HP), jnp.float32).at[0, :H].set(bm)

    nu = -(-user_emb.shape[0] // _BLKN)
    nm = -(-item_emb.shape[0] // _BLKN)
    uproj = _project(user_emb.T, wu, bup, nu)
    mproj = _project(item_emb.T, wm, bmp, nm)

    uidx = user_input.reshape(B // _CHUNK, _CHUNK)
    midx = movie_input.reshape(B // _CHUNK, _CHUNK)
    uhi, uoff = uidx >> 3, (uidx & (_G - 1)) * _HP
    mhi, moff = midx >> 3, (midx & (_G - 1)) * _HP

    z = _make_sc_zdot(B, H)(uhi, uoff, mhi, moff, uproj, mproj)
    return z.reshape(B, 1)
